# Initial kernel scaffold; baseline (speedup 1.0000x reference)
import jax
import jax.numpy as jnp
from jax.experimental import pallas as pl


def kernel(features, indices):
    raise NotImplementedError("write your pallas kernel here")


# speedup vs baseline: 1.6674x; 1.6674x over previous
import functools

import jax
import jax.numpy as jnp
import numpy as np
from jax.experimental import pallas as pl

UP = 2
VOL = UP * UP * UP

_ii, _jj, _kk = np.meshgrid(np.arange(UP), np.arange(UP), np.arange(UP), indexing="ij")
_OFF = np.stack((_ii, _jj, _kk), axis=-1).reshape(-1, 3)
_ORDER = np.argsort(_OFF[:, 0] * UP * UP + _OFF[:, 2] * UP + _OFF[:, 1], kind="stable")
_OFF = _OFF[_ORDER]

_SCALE = np.tile(np.array([1, UP, UP, UP], np.int32), VOL)
_BIAS = np.concatenate(
    [np.concatenate(([0], _OFF[v])).astype(np.int32) for v in range(VOL)]
)


def _body(f_ref, i_ref, fo_ref, io_ref):
    fo_ref[...] = f_ref[...]
    idx = i_ref[...]
    tiled = jnp.concatenate([idx] * VOL, axis=1)
    p = tiled.shape[0]
    l = jax.lax.broadcasted_iota(jnp.int32, (p, 4 * VOL), 1)
    c = l & 3
    v = l >> 2
    bias = jnp.where(
        c == 0, 0, jnp.where(c == 1, v >> 2, jnp.where(c == 2, v & 1, (v >> 1) & 1))
    )
    scale = jnp.where(c == 0, 1, UP)
    io_ref[...] = tiled * scale + bias


@jax.jit
def kernel(features, indices):
    n, c = features.shape
    out_c = c // VOL
    P = 5000
    grid = (n // P,)

    feats_out, idx_out = pl.pallas_call(
        _body,
        grid=grid,
        in_specs=[
            pl.BlockSpec((P, c), lambda i: (i, 0)),
            pl.BlockSpec((P, 4), lambda i: (i, 0)),
        ],
        out_specs=[
            pl.BlockSpec((P, c), lambda i: (i, 0)),
            pl.BlockSpec((P, 4 * VOL), lambda i: (i, 0)),
        ],
        out_shape=[
            jax.ShapeDtypeStruct((n, c), features.dtype),
            jax.ShapeDtypeStruct((n, 4 * VOL), jnp.int32),
        ],
    )(features, indices)

    return feats_out.reshape(n * VOL, out_c), idx_out.reshape(n * VOL, 4)


# speedup vs baseline: 3.3794x; 2.0267x over previous
import jax
import jax.numpy as jnp
from jax import lax
from jax.experimental import pallas as pl
from jax.experimental.pallas import tpu as pltpu

UP = 2
VOL = 8
BK = 16384


def _idx_body(i_ref, o_ref):
    x = i_ref[...]
    nk = BK // 128
    xr = x.reshape(4 * nk, 128).astype(jnp.float32)

    s_i = lax.broadcasted_iota(jnp.int32, (128, 8 * 128), 0)
    j_i = lax.broadcasted_iota(jnp.int32, (128, 8 * 128), 1)
    rep = (s_i == j_i // 8).astype(jnp.float32)

    y = lax.dot_general(
        xr, rep, (((1,), (0,)), ((), ())), preferred_element_type=jnp.float32
    )
    z = y.astype(jnp.int32).reshape(4, 8 * BK)

    c = lax.broadcasted_iota(jnp.int32, (4, 8 * BK), 0)
    v = lax.broadcasted_iota(jnp.int32, (4, 8 * BK), 1) & 7
    scale = jnp.where(c == 0, 1, UP)
    bias = jnp.where(
        c == 0, 0, jnp.where(c == 1, v >> 2, jnp.where(c == 2, v & 1, (v >> 1) & 1))
    )
    o_ref[...] = z * scale + bias


@jax.jit
def kernel(features, indices):
    n, nch = features.shape
    out_c = nch // VOL

    it = indices.T
    g = (n + BK - 1) // BK
    nip = pl.pallas_call(
        _idx_body,
        grid=(g,),
        in_specs=[pl.BlockSpec((4, BK), lambda i: (0, i))],
        out_specs=pl.BlockSpec((4, VOL * BK), lambda i: (0, i)),
        out_shape=jax.ShapeDtypeStruct((4, VOL * n), jnp.int32),
        compiler_params=pltpu.CompilerParams(
            dimension_semantics=("arbitrary",),
        ),
    )(it)

    return features.reshape(n * VOL, out_c), nip.T


# speedup vs baseline: 16.9413x; 5.0131x over previous
import functools

import jax
import jax.numpy as jnp
from jax import lax
from jax.experimental import pallas as pl
from jax.experimental.pallas import tpu as pltpu
from jax.experimental.pallas import tpu_sc as plsc

UP = 2
VOL = 8
BK = 16384

_NW = 32
_CM = 512
_N = 200000
_NFULL = _N // _CM
_REM = _N - _NFULL * _CM
_REMP = 384
_NSTEP = (_NFULL + 1 + _NW - 1) // _NW


def _idx_body(i_ref, o_ref):
    x = i_ref[...]
    nk = BK // 128
    xr = x.reshape(4 * nk, 128).astype(jnp.float32)

    s_i = lax.broadcasted_iota(jnp.int32, (128, 8 * 128), 0)
    j_i = lax.broadcasted_iota(jnp.int32, (128, 8 * 128), 1)
    rep = (s_i == j_i // 8).astype(jnp.float32)

    y = lax.dot_general(
        xr, rep, (((1,), (0,)), ((), ())), preferred_element_type=jnp.float32
    )
    z = y.astype(jnp.int32).reshape(4, 8 * BK)

    c = lax.broadcasted_iota(jnp.int32, (4, 8 * BK), 0)
    v = lax.broadcasted_iota(jnp.int32, (4, 8 * BK), 1) & 7
    scale = jnp.where(c == 0, 1, UP)
    bias = jnp.where(
        c == 0, 0, jnp.where(c == 1, v >> 2, jnp.where(c == 2, v & 1, (v >> 1) & 1))
    )
    o_ref[...] = z * scale + bias


def _sc_feats(ft_hbm, tail_hbm, fp_hbm, in_v, out_v):
    wid = lax.axis_index("s") * 2 + lax.axis_index("c")
    lane = lax.iota(jnp.int32, 16)
    i1b = lane >> 3

    def interleave(cols):
        for c in range(VOL):
            i0 = 8 * (lane & 7) + c

            @plsc.parallel_loop(0, VOL * cols // 16, unroll=8)
            def _(g):
                gat = plsc.load_gather(in_v, [i0, i1b + 2 * g])
                out_v[c, pl.ds(16 * g, 16)] = gat

    for step in range(_NSTEP):
        ch = wid + _NW * step

        @pl.when(ch < _NFULL)
        def _():
            m0 = pl.multiple_of(ch * _CM, _CM)
            pltpu.sync_copy(ft_hbm.at[:, pl.ds(m0, _CM)], in_v)
            interleave(_CM)
            o0 = pl.multiple_of(ch * (VOL * _CM), VOL * _CM)
            pltpu.sync_copy(out_v, fp_hbm.at[:, pl.ds(o0, VOL * _CM)])

        @pl.when(ch == _NFULL)
        def _():
            pltpu.sync_copy(tail_hbm, in_v.at[:, pl.ds(0, _REMP)])
            interleave(_REM)
            pltpu.sync_copy(
                out_v.at[:, pl.ds(0, VOL * _REM)],
                fp_hbm.at[:, pl.ds(VOL * _NFULL * _CM, VOL * _REM)],
            )


@jax.jit
def kernel(features, indices):
    n, nch = features.shape

    it = indices.T
    g = (n + BK - 1) // BK
    nip = pl.pallas_call(
        _idx_body,
        grid=(g,),
        in_specs=[pl.BlockSpec((4, BK), lambda i: (0, i))],
        out_specs=pl.BlockSpec((4, VOL * BK), lambda i: (0, i)),
        out_shape=jax.ShapeDtypeStruct((4, VOL * n), jnp.int32),
        compiler_params=pltpu.CompilerParams(
            dimension_semantics=("arbitrary",),
        ),
    )(it)

    ft = features.T
    tail = jnp.pad(features[_NFULL * _CM :, :].T, ((0, 0), (0, _REMP - _REM)))
    mesh = plsc.VectorSubcoreMesh(core_axis_name="c", subcore_axis_name="s")
    fpp = functools.partial(
        pl.kernel,
        mesh=mesh,
        out_type=jax.ShapeDtypeStruct((VOL, n * VOL), features.dtype),
        scratch_types=[
            pltpu.VMEM((nch, _CM), features.dtype),
            pltpu.VMEM((VOL, VOL * _CM), features.dtype),
        ],
        compiler_params=pltpu.CompilerParams(needs_layout_passes=False),
    )(_sc_feats)(ft, tail)

    return fpp.T, nip.T


# speedup vs baseline: 23.3576x; 1.3787x over previous
import functools

import jax
import jax.numpy as jnp
from jax import lax
from jax.experimental import pallas as pl
from jax.experimental.pallas import tpu as pltpu
from jax.experimental.pallas import tpu_sc as plsc

UP = 2
VOL = 8
BK = 16384

_NW = 32
_CM = 512
_N = 200000
_NFULL = _N // _CM
_REM = _N - _NFULL * _CM
_REMP = 384
_NSTEP = (_NFULL + 1 + _NW - 1) // _NW


def _idx_body(i_ref, o_ref):
    x = i_ref[...]
    nk = BK // 128
    xr = x.reshape(4 * nk, 128).astype(jnp.float32)

    s_i = lax.broadcasted_iota(jnp.int32, (128, 8 * 128), 0)
    j_i = lax.broadcasted_iota(jnp.int32, (128, 8 * 128), 1)
    rep = (s_i == j_i // 8).astype(jnp.float32)

    y = lax.dot_general(
        xr, rep, (((1,), (0,)), ((), ())), preferred_element_type=jnp.float32
    )
    z = y.astype(jnp.int32).reshape(4, 8 * BK)

    c = lax.broadcasted_iota(jnp.int32, (4, 8 * BK), 0)
    v = lax.broadcasted_iota(jnp.int32, (4, 8 * BK), 1) & 7
    scale = jnp.where(c == 0, 1, UP)
    bias = jnp.where(
        c == 0, 0, jnp.where(c == 1, v >> 2, jnp.where(c == 2, v & 1, (v >> 1) & 1))
    )
    o_ref[...] = z * scale + bias


def _sc_feats(ft_hbm, tail_hbm, fp_hbm, in_v, out_v, insem, outsem):
    wid = lax.axis_index("s") * 2 + lax.axis_index("c")
    lane = lax.iota(jnp.int32, 16)
    i1b = lane >> 3

    def in_full(s):
        ch = wid + _NW * s
        m0 = pl.multiple_of(ch * _CM, _CM)
        return pltpu.make_async_copy(
            ft_hbm.at[:, pl.ds(m0, _CM)], in_v.at[s % 2], insem.at[s % 2]
        )

    def in_tail(s):
        return pltpu.make_async_copy(
            tail_hbm, in_v.at[s % 2, :, pl.ds(0, _REMP)], insem.at[s % 2]
        )

    def out_full(s):
        ch = wid + _NW * s
        o0 = pl.multiple_of(ch * (VOL * _CM), VOL * _CM)
        return pltpu.make_async_copy(
            out_v.at[s % 2], fp_hbm.at[:, pl.ds(o0, VOL * _CM)], outsem.at[s % 2]
        )

    def out_tail(s):
        return pltpu.make_async_copy(
            out_v.at[s % 2, :, pl.ds(0, VOL * _REM)],
            fp_hbm.at[:, pl.ds(VOL * _NFULL * _CM, VOL * _REM)],
            outsem.at[s % 2],
        )

    def on_chunk(s, fn_full, fn_tail):
        ch = wid + _NW * s

        @pl.when(ch < _NFULL)
        def _():
            fn_full(s)

        @pl.when(ch == _NFULL)
        def _():
            fn_tail(s)

    def interleave(s, cols):
        for c in range(VOL):
            i0 = 8 * (lane & 7) + c

            @plsc.parallel_loop(0, VOL * cols // 16, unroll=8)
            def _(g):
                gat = plsc.load_gather(in_v.at[s % 2], [i0, i1b + 2 * g])
                out_v[s % 2, c, pl.ds(16 * g, 16)] = gat

    on_chunk(0, lambda s: in_full(s).start(), lambda s: in_tail(s).start())
    for s in range(_NSTEP):
        if s + 1 < _NSTEP:
            on_chunk(s + 1, lambda t: in_full(t).start(), lambda t: in_tail(t).start())
        on_chunk(s, lambda t: in_full(t).wait(), lambda t: in_tail(t).wait())
        if s >= 2:
            on_chunk(s - 2, lambda t: out_full(t).wait(), lambda t: out_tail(t).wait())
        on_chunk(s, lambda t: interleave(t, _CM), lambda t: interleave(t, _REM))
        on_chunk(s, lambda t: out_full(t).start(), lambda t: out_tail(t).start())
    for s in (_NSTEP - 2, _NSTEP - 1):
        on_chunk(s, lambda t: out_full(t).wait(), lambda t: out_tail(t).wait())


@jax.jit
def kernel(features, indices):
    n, nch = features.shape

    it = indices.T
    g = (n + BK - 1) // BK
    nip = pl.pallas_call(
        _idx_body,
        grid=(g,),
        in_specs=[pl.BlockSpec((4, BK), lambda i: (0, i))],
        out_specs=pl.BlockSpec((4, VOL * BK), lambda i: (0, i)),
        out_shape=jax.ShapeDtypeStruct((4, VOL * n), jnp.int32),
        compiler_params=pltpu.CompilerParams(
            dimension_semantics=("arbitrary",),
        ),
    )(it)

    ft = features.T
    tail = jnp.pad(features[_NFULL * _CM :, :].T, ((0, 0), (0, _REMP - _REM)))
    mesh = plsc.VectorSubcoreMesh(core_axis_name="c", subcore_axis_name="s")
    fpp = functools.partial(
        pl.kernel,
        mesh=mesh,
        out_type=jax.ShapeDtypeStruct((VOL, n * VOL), features.dtype),
        scratch_types=[
            pltpu.VMEM((2, nch, _CM), features.dtype),
            pltpu.VMEM((2, VOL, VOL * _CM), features.dtype),
            pltpu.SemaphoreType.DMA((2,)),
            pltpu.SemaphoreType.DMA((2,)),
        ],
        compiler_params=pltpu.CompilerParams(needs_layout_passes=False),
    )(_sc_feats)(ft, tail)

    return fpp.T, nip.T
